# baseline (speedup 1.0000x reference)
import functools

import jax
import jax.numpy as jnp
from jax import lax
from jax.experimental import pallas as pl
from jax.experimental.pallas import tpu as pltpu

T, D, H, E, A = 4096, 768, 64, 14, 128
TILE_T = 1024
K = 3
NCHUNK = 2
CE = E // NCHUNK


def _moe_kernel(x_ref, wg1_ref, bg1_ref, wg2_ref, bg2_ref, wer_ref, be_ref,
                y_ref, comb_ref):
    c = pl.program_id(1)
    x = x_ref[...]

    @pl.when(c == 0)
    def _gating():
        h = jnp.maximum(
            jnp.dot(x, wg1_ref[...], preferred_element_type=jnp.float32)
            + bg1_ref[...], 0.0)
        logits_t = lax.dot_general(
            wg2_ref[...], h, (((0,), (1,)), ((), ())),
            preferred_element_type=jnp.float32) + bg2_ref[...].T
        m = jnp.max(logits_t, axis=0, keepdims=True)
        ex = jnp.exp(logits_t - m)
        w = ex / jnp.sum(ex, axis=0, keepdims=True)

        row = lax.broadcasted_iota(jnp.int32, w.shape, 0)
        mask = jnp.zeros(w.shape, jnp.bool_)
        for _ in range(K):
            cand = jnp.where(mask, -1.0, w)
            mx = jnp.max(cand, axis=0, keepdims=True)
            first = jnp.min(jnp.where(cand == mx, row, E), axis=0,
                            keepdims=True)
            mask = mask | (row == first)
        combine = jnp.where(mask, w, 0.0).T
        for cc in range(NCHUNK):
            comb_ref[cc] = combine[:, cc * CE:(cc + 1) * CE]
        y_ref[...] = jnp.dot(combine, be_ref[...],
                             preferred_element_type=jnp.float32)

    combine_c = comb_ref[c]
    acc0 = y_ref[...]
    acc1 = jnp.zeros_like(acc0)
    for e in range(CE):
        xe = jnp.dot(x, wer_ref[e], preferred_element_type=jnp.float32)
        if e % 2 == 0:
            acc0 = acc0 + combine_c[:, e:e + 1] * xe
        else:
            acc1 = acc1 + combine_c[:, e:e + 1] * xe
    y_ref[...] = acc0 + acc1


@functools.partial(jax.jit, static_argnames=())
def _run(x, Wg1, bg1, Wg2, bg2, We, be):
    grid = (T // TILE_T, NCHUNK)
    return pl.pallas_call(
        _moe_kernel,
        grid=grid,
        in_specs=[
            pl.BlockSpec((TILE_T, D), lambda i, c: (i, 0)),
            pl.BlockSpec((D, H), lambda i, c: (0, 0)),
            pl.BlockSpec((1, H), lambda i, c: (0, 0)),
            pl.BlockSpec((H, E), lambda i, c: (0, 0)),
            pl.BlockSpec((1, E), lambda i, c: (0, 0)),
            pl.BlockSpec((CE, D, A), lambda i, c: (c, 0, 0)),
            pl.BlockSpec((E, A), lambda i, c: (0, 0)),
        ],
        out_specs=pl.BlockSpec((TILE_T, A), lambda i, c: (i, 0)),
        out_shape=jax.ShapeDtypeStruct((T, A), jnp.float32),
        scratch_shapes=[pltpu.VMEM((NCHUNK, TILE_T, CE), jnp.float32)],
    )(x, Wg1, bg1, Wg2, bg2, We, be)


def kernel(x, Wg1, bg1, Wg2, bg2, We, be, top_k):
    return _run(x, Wg1, bg1.reshape(1, H), Wg2, bg2.reshape(1, E), We, be)


# speedup vs baseline: 1.0912x; 1.0912x over previous
import functools

import jax
import jax.numpy as jnp
from jax import lax
from jax.experimental import pallas as pl
from jax.experimental.pallas import tpu as pltpu

T, D, H, E, A = 4096, 768, 64, 14, 128
TILE_T = 1024
K = 3


def _moe_kernel(x_ref, wg1_ref, bg1_ref, wg2_ref, bg2_ref, wer_ref, be_ref,
                y_ref):
    x = x_ref[...]
    h = jnp.maximum(
        jnp.dot(x, wg1_ref[...], preferred_element_type=jnp.float32)
        + bg1_ref[...], 0.0)
    logits_t = lax.dot_general(
        wg2_ref[...], h, (((0,), (1,)), ((), ())),
        preferred_element_type=jnp.float32) + bg2_ref[...].T
    m = jnp.max(logits_t, axis=0, keepdims=True)
    ex = jnp.exp(logits_t - m)
    w = ex / jnp.sum(ex, axis=0, keepdims=True)

    row = lax.broadcasted_iota(jnp.int32, w.shape, 0)
    mask = jnp.zeros(w.shape, jnp.bool_)
    for _ in range(K):
        cand = jnp.where(mask, -1.0, w)
        mx = jnp.max(cand, axis=0, keepdims=True)
        first = jnp.min(jnp.where(cand == mx, row, E), axis=0, keepdims=True)
        mask = mask | (row == first)
    combine = jnp.where(mask, w, 0.0).T

    acc0 = jnp.dot(combine, be_ref[...], preferred_element_type=jnp.float32)
    acc1 = jnp.zeros_like(acc0)
    for e in range(E):
        xe = jnp.dot(x, wer_ref[e], preferred_element_type=jnp.float32)
        if e % 2 == 0:
            acc0 = acc0 + combine[:, e:e + 1] * xe
        else:
            acc1 = acc1 + combine[:, e:e + 1] * xe
    y_ref[...] = acc0 + acc1


@functools.partial(jax.jit, static_argnames=())
def _run(x, Wg1, bg1, Wg2, bg2, We, be):
    grid = (T // TILE_T,)
    return pl.pallas_call(
        _moe_kernel,
        grid=grid,
        in_specs=[
            pl.BlockSpec((TILE_T, D), lambda i: (i, 0)),
            pl.BlockSpec((D, H), lambda i: (0, 0)),
            pl.BlockSpec((1, H), lambda i: (0, 0)),
            pl.BlockSpec((H, E), lambda i: (0, 0)),
            pl.BlockSpec((1, E), lambda i: (0, 0)),
            pl.BlockSpec((E, D, A), lambda i: (0, 0, 0)),
            pl.BlockSpec((E, A), lambda i: (0, 0)),
        ],
        out_specs=pl.BlockSpec((TILE_T, A), lambda i: (i, 0)),
        out_shape=jax.ShapeDtypeStruct((T, A), jnp.float32),
        compiler_params=pltpu.CompilerParams(
            fuse_transposed_lhs_in_matmul=True),
    )(x, Wg1, bg1, Wg2, bg2, We, be)


def kernel(x, Wg1, bg1, Wg2, bg2, We, be, top_k):
    return _run(x, Wg1, bg1.reshape(1, H), Wg2, bg2.reshape(1, E), We, be)
